# Initial kernel scaffold; baseline (speedup 1.0000x reference)
import jax
import jax.numpy as jnp
from jax.experimental import pallas as pl


def kernel(x, Wq, Wk, Wv, Wo):
    raise NotImplementedError("write your pallas kernel here")


# speedup vs baseline: 1.2563x; 1.2563x over previous
import functools

import jax
import jax.numpy as jnp
from jax.experimental import pallas as pl

_H = 16
_CHUNK = 256
_TOP_K = 5
_NEG = -1e9


def _mm_t_kernel(a_ref, w_ref, o_ref):
    o_ref[...] = jax.lax.dot_general(
        a_ref[...], w_ref[...], (((1,), (1,)), ((), ())),
        preferred_element_type=jnp.float32)


def _matmul_t(a, w, bm, bn):
    M, K = a.shape
    N = w.shape[0]
    return pl.pallas_call(
        _mm_t_kernel,
        grid=(M // bm, N // bn),
        in_specs=[
            pl.BlockSpec((bm, K), lambda i, j: (i, 0)),
            pl.BlockSpec((bn, K), lambda i, j: (j, 0)),
        ],
        out_specs=pl.BlockSpec((bm, bn), lambda i, j: (i, j)),
        out_shape=jax.ShapeDtypeStruct((M, N), jnp.float32),
    )(a, w)


def _attn_one_head(Q, K, V, seq, hd, scale):
    nc = seq // _CHUNK

    ck_rows = [
        jnp.sum(K[c * _CHUNK:(c + 1) * _CHUNK, :], axis=0, keepdims=True)
        * (1.0 / _CHUNK)
        for c in range(nc)
    ]
    ck = jnp.concatenate(ck_rows, axis=0)

    sims = jax.lax.dot_general(
        Q, ck, (((1,), (1,)), ((), ())),
        preferred_element_type=jnp.float32) * scale

    sel_cols = []
    for c in range(nc):
        sc = sims[:, c:c + 1]
        rank = jnp.zeros((seq, 1), jnp.int32)
        for cp in range(nc):
            if cp == c:
                continue
            sp = sims[:, cp:cp + 1]
            gt = sp > sc
            if cp < c:
                gt = jnp.logical_or(gt, sp == sc)
            rank = rank + gt.astype(jnp.int32)
        sel_cols.append(rank < _TOP_K)

    mean_v = jnp.sum(V, axis=0, keepdims=True) * (1.0 / seq)

    out_chunks = []
    for cq in range(nc):
        q0 = cq * _CHUNK
        kend = (cq + 1) * _CHUNK
        Qb = Q[q0:q0 + _CHUNK, :]
        scores = jax.lax.dot_general(
            Qb, K[:kend, :], (((1,), (1,)), ((), ())),
            preferred_element_type=jnp.float32) * scale

        parts = []
        any_sel = None
        for c in range(cq + 1):
            m = sel_cols[c][q0:q0 + _CHUNK, :]
            any_sel = m if any_sel is None else jnp.logical_or(any_sel, m)
            mb = jnp.broadcast_to(m, (_CHUNK, _CHUNK))
            if c == cq:
                ri = jax.lax.broadcasted_iota(jnp.int32, (_CHUNK, _CHUNK), 0)
                ci = jax.lax.broadcasted_iota(jnp.int32, (_CHUNK, _CHUNK), 1)
                mb = jnp.logical_and(mb, ri >= ci)
            parts.append(mb)
        mask = jnp.concatenate(parts, axis=1)

        s = jnp.where(mask, scores, _NEG)
        mx = jnp.max(s, axis=1, keepdims=True)
        p = jnp.exp(s - mx)
        dn = jnp.sum(p, axis=1, keepdims=True)
        out = jnp.dot(p, V[:kend, :], preferred_element_type=jnp.float32) / dn
        out = jnp.where(any_sel, out, jnp.broadcast_to(mean_v, (_CHUNK, hd)))
        out_chunks.append(out)
    return jnp.concatenate(out_chunks, axis=0)


def _attn_kernel(q_ref, k_ref, v_ref, o_ref, *, seq, hd, hpp, scale):
    outs = []
    for sh in range(hpp):
        c0 = sh * hd
        outs.append(_attn_one_head(
            q_ref[:, c0:c0 + hd], k_ref[:, c0:c0 + hd],
            v_ref[:, c0:c0 + hd], seq, hd, scale))
    o_ref[...] = jnp.concatenate(outs, axis=1)


def _attention(q, k, v, scale):
    seq, d = q.shape
    hd = d // _H
    hpp = 2
    bw = hpp * hd
    kern = functools.partial(_attn_kernel, seq=seq, hd=hd, hpp=hpp,
                             scale=scale)
    return pl.pallas_call(
        kern,
        grid=(_H // hpp,),
        in_specs=[pl.BlockSpec((seq, bw), lambda h: (0, h))] * 3,
        out_specs=pl.BlockSpec((seq, bw), lambda h: (0, h)),
        out_shape=jax.ShapeDtypeStruct((seq, d), jnp.float32),
    )(q, k, v)


def kernel(x, Wq, Wk, Wv, Wo):
    b, s, d = x.shape
    hd = d // _H
    scale = hd ** -0.5
    x2d = x.reshape(b * s, d)
    q = _matmul_t(x2d, Wq, 256, 512)
    k = _matmul_t(x2d, Wk, 256, 512)
    v = _matmul_t(x2d, Wv, 256, 512)
    attn = _attention(q, k, v, scale)
    out = _matmul_t(attn, Wo, 256, 512)
    return out.reshape(b, s, d)
